# baseline (speedup 1.0000x reference)
import functools
import jax
import jax.numpy as jnp
from jax import lax
from jax.experimental import pallas as pl
from jax.experimental.pallas import tpu as pltpu
from jax.experimental.pallas import tpu_sc as plsc


VOCAB_PAD = 1024
WORDS = VOCAB_PAD // 4
LANES = 16
NC, NS = 2, 16
NW = NC * NS


def _sc_hist_body(idx_hbm, out_hbm, idx_v, cnt_v, sem, *, rows_per_w, hist):
    wid = lax.axis_index("s") * NC + lax.axis_index("c")
    base = wid * rows_per_w

    stage = pltpu.async_copy(idx_hbm.at[pl.ds(base, rows_per_w)], idx_v, sem)

    zero16 = jnp.zeros((LANES,), jnp.int32)
    groups_per_row = WORDS // LANES

    @plsc.parallel_loop(0, rows_per_w, unroll=2)
    def _(r):
        for j in range(groups_per_row):
            cnt_v[r, pl.ds(j * LANES, LANES)] = zero16

    stage.wait()

    nfull = hist // LANES
    tail = hist - nfull * LANES
    one = jnp.full((LANES,), 1, jnp.int32)
    lane = lax.iota(jnp.int32, LANES)

    def scatter_group(tile_base, vidx, mask):
        w = vidx & 255
        flat = tile_base + w
        val = one << ((vidx >> 8) << 3)
        plsc.addupdate_scatter(cnt_v, [flat >> 8, flat & 255], val, mask=mask)

    @plsc.parallel_loop(0, rows_per_w, unroll=2)
    def _(r):
        tile_base = r * WORDS
        for j in range(nfull):
            vidx = idx_v[r, pl.ds(j * LANES, LANES)]
            scatter_group(tile_base, vidx, None)
        if tail:
            vidx = idx_v[r, pl.ds(hist - LANES, LANES)]
            scatter_group(tile_base, vidx, lane >= (LANES - tail))

    pltpu.sync_copy(cnt_v, out_hbm.at[pl.ds(base, rows_per_w), :])


def _sc_hist(inputs, *, b, hist):
    rows_per_w = b // NW
    mesh = plsc.VectorSubcoreMesh(core_axis_name="c", subcore_axis_name="s")
    body = functools.partial(_sc_hist_body, rows_per_w=rows_per_w, hist=hist)
    f = pl.kernel(
        body,
        out_type=jax.ShapeDtypeStruct((b, WORDS), jnp.int32),
        mesh=mesh,
        scratch_types=[
            pltpu.VMEM((rows_per_w, hist), jnp.int32),
            pltpu.VMEM((rows_per_w, WORDS), jnp.int32),
            pltpu.SemaphoreType.DMA,
        ],
        compiler_params=pltpu.CompilerParams(
            needs_layout_passes=False, use_tc_tiling_on_sc=True),
    )
    return f(inputs)


def _final_kernel(packed_ref, emb_ref, w_ref, b_ref, out_ref, m_ref, *,
                  inv_l, vocab):
    @pl.when(pl.program_id(0) == 0)
    def _():
        m_ref[...] = jnp.zeros_like(m_ref)
        m_ref[:vocab, :] = lax.dot_general(
            emb_ref[...], w_ref[...],
            dimension_numbers=(((1,), (1,)), ((), ())),
            preferred_element_type=jnp.float32,
        ) * inv_l

    packed = packed_ref[...]
    tb = packed.shape[0]
    z = jnp.zeros((tb, b_ref.shape[1]), jnp.float32) + b_ref[...]
    for p in range(4):
        plane = ((packed >> (8 * p)) & 255).astype(jnp.float32)
        z = z + lax.dot_general(
            plane, m_ref[pl.ds(256 * p, 256), :],
            dimension_numbers=(((1,), (0,)), ((), ())),
            preferred_element_type=jnp.float32,
        )
    zmax = jnp.max(z, axis=1, keepdims=True)
    s = z - zmax
    lse = jnp.log(jnp.sum(jnp.exp(s), axis=1, keepdims=True))
    out_ref[...] = s - lse


def kernel(inputs, emb, W, b):
    B, L = inputs.shape
    V, D = emb.shape
    C = W.shape[0]

    packed = _sc_hist(inputs, b=B, hist=L)

    TB = 1024
    b2 = b.reshape(1, C)
    out = pl.pallas_call(
        functools.partial(_final_kernel, inv_l=1.0 / L, vocab=V),
        grid=(B // TB,),
        in_specs=[
            pl.BlockSpec((TB, WORDS), lambda i: (i, 0)),
            pl.BlockSpec((V, D), lambda i: (0, 0)),
            pl.BlockSpec((C, D), lambda i: (0, 0)),
            pl.BlockSpec((1, C), lambda i: (0, 0)),
        ],
        out_specs=pl.BlockSpec((TB, C), lambda i: (i, 0)),
        out_shape=jax.ShapeDtypeStruct((B, C), jnp.float32),
        scratch_shapes=[pltpu.VMEM((VOCAB_PAD, C), jnp.float32)],
    )(packed, emb, W, b2)

    return out
